# baseline (speedup 1.0000x reference)
import functools

import jax
import jax.numpy as jnp
from jax import lax
from jax.experimental import pallas as pl
from jax.experimental.pallas import tpu as pltpu
from jax.experimental.pallas import tpu_sc as plsc

N_NODES = 10000
N_PAD = 10112
E = 640000
NC, NS = 2, 16
NW = NC * NS
CHUNK = 125
N_CHUNKS = 160
H = 4
NBLK = N_CHUNKS // H
ROWS_T = N_PAD // NS

_MESH = plsc.VectorSubcoreMesh(
    core_axis_name="c", subcore_axis_name="s", num_cores=NC, num_subcores=NS)
_SC_PARAMS = pltpu.CompilerParams(
    needs_layout_passes=False, use_tc_tiling_on_sc=False)


def _make_prop(w):
    def body(g_hbm, src_hbm, dst_hbm, z_hbm, out_hbm,
             src_v, dst_v, rows_v, bb_v, g_sh, acc_sh, sem_g, sem_s):
        c = lax.axis_index("c")
        s = lax.axis_index("s")
        wid = c * NS + s
        r0 = s * ROWS_T
        pltpu.sync_copy(z_hbm.at[pl.ds(r0, ROWS_T)], bb_v)
        pltpu.sync_copy(bb_v, acc_sh.at[pl.ds(r0, ROWS_T)])
        pltpu.sync_copy(g_hbm.at[pl.ds(r0, ROWS_T)], bb_v)
        pltpu.sync_copy(bb_v, g_sh.at[pl.ds(r0, ROWS_T)])
        pltpu.sync_copy(src_hbm.at[wid], src_v)
        pltpu.sync_copy(dst_hbm.at[wid], dst_v)
        plsc.subcore_barrier()

        def issue_gathers(blk, half):
            for b in range(H):
                pltpu.async_copy(g_sh.at[src_v.at[blk * H + b]],
                                 rows_v.at[half * H + b], sem_g)

        def issue_scatters(blk, half):
            for b in range(H):
                pltpu.async_copy(rows_v.at[half * H + b],
                                 acc_sh.at[dst_v.at[blk * H + b]],
                                 sem_s, add=True)

        def drain(sem, n):
            for _ in range(n):
                pltpu.make_async_copy(g_sh.at[src_v.at[0]],
                                      rows_v.at[0], sem).wait()

        issue_gathers(0, 0)

        def loop(p, carry):
            blk_a = 2 * p
            blk_b = 2 * p + 1

            @pl.when(p >= 1)
            def _():
                drain(sem_s, H)

            issue_gathers(blk_b, 1)
            drain(sem_g, H)
            issue_scatters(blk_a, 0)
            drain(sem_s, H)

            @pl.when(blk_a + 2 < NBLK)
            def _():
                issue_gathers(blk_a + 2, 0)

            drain(sem_g, H)
            issue_scatters(blk_b, 1)
            return carry

        lax.fori_loop(0, NBLK // 2, loop, 0)
        drain(sem_s, H)
        plsc.subcore_barrier()
        pltpu.sync_copy(acc_sh.at[pl.ds(r0, ROWS_T)], bb_v)
        pltpu.sync_copy(bb_v, out_hbm.at[c, pl.ds(r0, ROWS_T)])

    return functools.partial(
        pl.kernel,
        out_type=jax.ShapeDtypeStruct((NC, N_PAD, w), jnp.float32),
        mesh=_MESH,
        compiler_params=_SC_PARAMS,
        scratch_types=[
            pltpu.VMEM((N_CHUNKS, CHUNK), jnp.int32),
            pltpu.VMEM((N_CHUNKS, CHUNK), jnp.int32),
            pltpu.VMEM((2 * H, CHUNK, w), jnp.float32),
            pltpu.VMEM((ROWS_T, w), jnp.float32),
            pltpu.VMEM_SHARED((N_PAD, w), jnp.float32),
            pltpu.VMEM_SHARED((N_PAD, w), jnp.float32),
            pltpu.SemaphoreType.DMA,
            pltpu.SemaphoreType.DMA,
        ],
    )(body)


_prop16 = _make_prop(16)
_prop8 = _make_prop(8)
_prop1 = _make_prop(1)


def _dec_body(x_ref, w_ref, b_ref, o_ref):
    acc = jnp.dot(x_ref[...], w_ref[...], preferred_element_type=jnp.float32)
    o_ref[...] = jnp.maximum(acc + b_ref[...], 0.0)


def _tc_decoder(x, W_dec, b_dec):
    bk = 6400
    grid = W_dec.shape[1] // bk
    return pl.pallas_call(
        _dec_body,
        grid=(grid,),
        in_specs=[
            pl.BlockSpec((1, 64), lambda i: (0, 0)),
            pl.BlockSpec((64, bk), lambda i: (0, i)),
            pl.BlockSpec((1, bk), lambda i: (0, i)),
        ],
        out_specs=pl.BlockSpec((1, bk), lambda i: (0, i)),
        out_shape=jax.ShapeDtypeStruct((1, W_dec.shape[1]), jnp.float32),
    )(x, W_dec, b_dec.reshape(1, -1))


def _norm_body(dacc_ref, h_ref, w_ref, g_ref, dinv_ref):
    deg = dacc_ref[0] + dacc_ref[1] + 1.0
    dinv = lax.rsqrt(deg)
    dinv_ref[...] = dinv
    hw = jnp.dot(h_ref[...], w_ref[...], preferred_element_type=jnp.float32)
    g_ref[...] = dinv * hw


def _tc_norm(dacc, h0p, W4):
    return pl.pallas_call(
        _norm_body,
        out_shape=[
            jax.ShapeDtypeStruct((N_PAD, 16), jnp.float32),
            jax.ShapeDtypeStruct((N_PAD, 1), jnp.float32),
        ],
    )(dacc, h0p, W4)


def _mid_body(acc_ref, g_ref, dinv_ref, b_ref, w_ref, o_ref):
    dinv = dinv_ref[...]
    pre = dinv * (acc_ref[0] + acc_ref[1] + g_ref[...]) + b_ref[...]
    h = jnp.maximum(pre, 0.0)
    o_ref[...] = dinv * jnp.dot(h, w_ref[...], preferred_element_type=jnp.float32)


def _tc_mid(acc, g, dinv, b, W, w_out):
    return pl.pallas_call(
        _mid_body,
        out_shape=jax.ShapeDtypeStruct((N_PAD, w_out), jnp.float32),
    )(acc, g, dinv, b.reshape(1, -1), W)


def _fin_body(acc_ref, g_ref, dinv_ref, b_ref, o_ref):
    pre = dinv_ref[...] * (acc_ref[0] + acc_ref[1] + g_ref[...]) + b_ref[...]
    o_ref[...] = jax.nn.sigmoid(pre)


def _tc_final(acc, g2, dinv, b6):
    return pl.pallas_call(
        _fin_body,
        out_shape=jax.ShapeDtypeStruct((N_PAD, 1), jnp.float32),
    )(acc, g2, dinv, b6.reshape(1, 1))


def kernel(x, edge_index, W_dec, b_dec, W4, b4, W5, b5, W6, b6):
    src3 = edge_index[0].reshape(NW, N_CHUNKS, CHUNK)
    dst3 = edge_index[1].reshape(NW, N_CHUNKS, CHUNK)

    ones1 = jnp.ones((N_PAD, 1), jnp.float32)
    z1 = jnp.zeros((N_PAD, 1), jnp.float32)
    z8 = jnp.zeros((N_PAD, 8), jnp.float32)
    z16 = jnp.zeros((N_PAD, 16), jnp.float32)

    dacc = _prop1(ones1, src3, dst3, z1)
    h0 = _tc_decoder(x, W_dec, b_dec)
    h0p = jnp.pad(h0.reshape(N_NODES, 16), ((0, N_PAD - N_NODES), (0, 0)))

    g0, dinv = _tc_norm(dacc, h0p, W4)

    acc1 = _prop16(g0, src3, dst3, z16)
    g1 = _tc_mid(acc1, g0, dinv, b4, W5, 8)

    acc2 = _prop8(g1, src3, dst3, z8)
    g2 = _tc_mid(acc2, g1, dinv, b5, W6, 1)

    acc3 = _prop1(g2, src3, dst3, z1)
    out = _tc_final(acc3, g2, dinv, b6)
    return out[:N_NODES, 0].reshape(1, N_NODES)


# speedup vs baseline: 1.0724x; 1.0724x over previous
import functools

import jax
import jax.numpy as jnp
from jax import lax
from jax.experimental import pallas as pl
from jax.experimental.pallas import tpu as pltpu
from jax.experimental.pallas import tpu_sc as plsc

N_NODES = 10000
N_PAD = 10112
E = 640000
NC, NS = 2, 16
NW = NC * NS
CHUNK = 125
N_CHUNKS = 160
H = 4
NBLK = N_CHUNKS // H
ROWS_T = N_PAD // NS

_MESH = plsc.VectorSubcoreMesh(
    core_axis_name="c", subcore_axis_name="s", num_cores=NC, num_subcores=NS)
_SC_PARAMS = pltpu.CompilerParams(
    needs_layout_passes=False, use_tc_tiling_on_sc=False)


def _make_prop(w):
    def body(g_hbm, e_hbm, z_hbm, out_hbm,
             src_v, dst_v, rows_v, bb_v, g_sh, acc_sh, sem_g, sem_s):
        c = lax.axis_index("c")
        s = lax.axis_index("s")
        wid = c * NS + s
        r0 = s * ROWS_T
        pltpu.sync_copy(z_hbm.at[pl.ds(r0, ROWS_T)], bb_v)
        pltpu.sync_copy(bb_v, acc_sh.at[pl.ds(r0, ROWS_T)])
        pltpu.sync_copy(g_hbm.at[pl.ds(r0, ROWS_T)], bb_v)
        pltpu.sync_copy(bb_v, g_sh.at[pl.ds(r0, ROWS_T)])
        pltpu.sync_copy(e_hbm.at[0, wid], src_v)
        pltpu.sync_copy(e_hbm.at[1, wid], dst_v)
        plsc.subcore_barrier()

        def issue_gathers(blk, half):
            for b in range(H):
                pltpu.async_copy(g_sh.at[src_v.at[blk * H + b]],
                                 rows_v.at[half * H + b], sem_g)

        def issue_scatters(blk, half):
            for b in range(H):
                pltpu.async_copy(rows_v.at[half * H + b],
                                 acc_sh.at[dst_v.at[blk * H + b]],
                                 sem_s, add=True)

        def drain(sem, n):
            for _ in range(n):
                pltpu.make_async_copy(g_sh.at[src_v.at[0]],
                                      rows_v.at[0], sem).wait()

        issue_gathers(0, 0)

        def loop(p, carry):
            blk_a = 2 * p
            blk_b = 2 * p + 1

            @pl.when(p >= 1)
            def _():
                drain(sem_s, H)

            issue_gathers(blk_b, 1)
            drain(sem_g, H)
            issue_scatters(blk_a, 0)
            drain(sem_s, H)

            @pl.when(blk_a + 2 < NBLK)
            def _():
                issue_gathers(blk_a + 2, 0)

            drain(sem_g, H)
            issue_scatters(blk_b, 1)
            return carry

        lax.fori_loop(0, NBLK // 2, loop, 0)
        drain(sem_s, H)
        plsc.subcore_barrier()
        pltpu.sync_copy(acc_sh.at[pl.ds(r0, ROWS_T)], bb_v)
        pltpu.sync_copy(bb_v, out_hbm.at[c, pl.ds(r0, ROWS_T)])

    return functools.partial(
        pl.kernel,
        out_type=jax.ShapeDtypeStruct((NC, N_PAD, w), jnp.float32),
        mesh=_MESH,
        compiler_params=_SC_PARAMS,
        scratch_types=[
            pltpu.VMEM((N_CHUNKS, CHUNK), jnp.int32),
            pltpu.VMEM((N_CHUNKS, CHUNK), jnp.int32),
            pltpu.VMEM((2 * H, CHUNK, w), jnp.float32),
            pltpu.VMEM((ROWS_T, w), jnp.float32),
            pltpu.VMEM_SHARED((N_PAD, w), jnp.float32),
            pltpu.VMEM_SHARED((N_PAD, w), jnp.float32),
            pltpu.SemaphoreType.DMA,
            pltpu.SemaphoreType.DMA,
        ],
    )(body)


_prop16 = _make_prop(16)
_prop8 = _make_prop(8)
_prop1 = _make_prop(1)


def _dec_body(x_ref, w_ref, b_ref, o_ref):
    acc = jnp.dot(x_ref[...], w_ref[...], preferred_element_type=jnp.float32)
    o_ref[...] = jnp.maximum(acc + b_ref[...], 0.0)


def _tc_decoder(x, W_dec, b_dec):
    bk = 6400
    grid = W_dec.shape[1] // bk
    return pl.pallas_call(
        _dec_body,
        grid=(grid,),
        in_specs=[
            pl.BlockSpec((1, 64), lambda i: (0, 0)),
            pl.BlockSpec((64, bk), lambda i: (0, i)),
            pl.BlockSpec((1, bk), lambda i: (0, i)),
        ],
        out_specs=pl.BlockSpec((1, bk), lambda i: (0, i)),
        out_shape=jax.ShapeDtypeStruct((1, W_dec.shape[1]), jnp.float32),
    )(x, W_dec, b_dec.reshape(1, -1))


def _norm_body(dacc_ref, h_ref, w_ref, g_ref, dinv_ref):
    deg = dacc_ref[0] + dacc_ref[1] + 1.0
    dinv = lax.rsqrt(deg)
    dinv_ref[...] = dinv
    hw = jnp.dot(h_ref[...], w_ref[...], preferred_element_type=jnp.float32)
    g_ref[...] = dinv * hw


def _tc_norm(dacc, h0p, W4):
    return pl.pallas_call(
        _norm_body,
        out_shape=[
            jax.ShapeDtypeStruct((N_PAD, 16), jnp.float32),
            jax.ShapeDtypeStruct((N_PAD, 1), jnp.float32),
        ],
    )(dacc, h0p, W4)


def _mid_body(acc_ref, g_ref, dinv_ref, b_ref, w_ref, o_ref):
    dinv = dinv_ref[...]
    pre = dinv * (acc_ref[0] + acc_ref[1] + g_ref[...]) + b_ref[...]
    h = jnp.maximum(pre, 0.0)
    o_ref[...] = dinv * jnp.dot(h, w_ref[...], preferred_element_type=jnp.float32)


def _tc_mid(acc, g, dinv, b, W, w_out):
    return pl.pallas_call(
        _mid_body,
        out_shape=jax.ShapeDtypeStruct((N_PAD, w_out), jnp.float32),
    )(acc, g, dinv, b.reshape(1, -1), W)


def _fin_body(acc_ref, g_ref, dinv_ref, b_ref, o_ref):
    pre = dinv_ref[...] * (acc_ref[0] + acc_ref[1] + g_ref[...]) + b_ref[...]
    o_ref[...] = jax.nn.sigmoid(pre)


def _tc_final(acc, g2, dinv, b6):
    return pl.pallas_call(
        _fin_body,
        out_shape=jax.ShapeDtypeStruct((N_PAD, 1), jnp.float32),
    )(acc, g2, dinv, b6.reshape(1, 1))


def kernel(x, edge_index, W_dec, b_dec, W4, b4, W5, b5, W6, b6):
    e4 = edge_index.reshape(2, NW, N_CHUNKS, CHUNK)

    ones1 = jnp.ones((N_PAD, 1), jnp.float32)
    z1 = jnp.zeros((N_PAD, 1), jnp.float32)
    z8 = jnp.zeros((N_PAD, 8), jnp.float32)
    z16 = jnp.zeros((N_PAD, 16), jnp.float32)

    dacc = _prop1(ones1, e4, z1)
    h0 = _tc_decoder(x, W_dec, b_dec)
    h0p = jnp.pad(h0.reshape(N_NODES, 16), ((0, N_PAD - N_NODES), (0, 0)))

    g0, dinv = _tc_norm(dacc, h0p, W4)

    acc1 = _prop16(g0, e4, z16)
    g1 = _tc_mid(acc1, g0, dinv, b4, W5, 8)

    acc2 = _prop8(g1, e4, z8)
    g2 = _tc_mid(acc2, g1, dinv, b5, W6, 1)

    acc3 = _prop1(g2, e4, z1)
    out = _tc_final(acc3, g2, dinv, b6)
    return out[:N_NODES, 0].reshape(1, N_NODES)
